# Initial kernel scaffold; baseline (speedup 1.0000x reference)
import jax
import jax.numpy as jnp
from jax.experimental import pallas as pl


def kernel(x, edge_index, W, b):
    raise NotImplementedError("write your pallas kernel here")


# speedup vs baseline: 11.7209x; 11.7209x over previous
import functools

import jax
import jax.numpy as jnp
from jax import lax
from jax.experimental import pallas as pl
from jax.experimental.pallas import tpu as pltpu
from jax.experimental.pallas import tpu_sc as plsc

N_NODES = 10000
N_EDGES = 320000
D = 128

NC = 2
NS = 16
NW = NC * NS
EPW = N_EDGES // NW
CH = 80
NCHUNK = EPW // CH
RPT = 640
RPT_LAST = N_NODES - 15 * RPT
ZR = 40

_mesh = plsc.VectorSubcoreMesh(
    core_axis_name="c", subcore_axis_name="s", num_cores=NC, num_subcores=NS
)


@functools.partial(
    pl.kernel,
    mesh=_mesh,
    out_type=(
        jax.ShapeDtypeStruct((NC, N_NODES, D), jnp.float32),
        jax.ShapeDtypeStruct((NW * N_NODES,), jnp.int32),
    ),
    scratch_types=[
        pltpu.VMEM((CH,), jnp.int32),
        pltpu.VMEM((CH,), jnp.int32),
        pltpu.VMEM((CH,), jnp.int32),
        pltpu.VMEM((CH,), jnp.int32),
        pltpu.VMEM((CH, D), jnp.float32),
        pltpu.VMEM((CH, D), jnp.float32),
        pltpu.VMEM((ZR, D), jnp.float32),
        pltpu.VMEM((N_NODES,), jnp.int32),
        pltpu.VMEM_SHARED((N_NODES, D), jnp.float32),
        pltpu.SemaphoreType.DMA,
        pltpu.SemaphoreType.DMA,
        pltpu.SemaphoreType.DMA,
        pltpu.SemaphoreType.DMA,
        pltpu.SemaphoreType.DMA,
        pltpu.SemaphoreType.DMA,
    ],
)
def _aggregate(row_hbm, col_hbm, x_hbm, acc_out, cnt_out,
               colv0, colv1, rowv0, rowv1, rows0, rows1, z, hist,
               acc_s, semI0, semI1, semG0, semG1, semS0, semS1):
    c = lax.axis_index("c")
    s = lax.axis_index("s")
    wid = c * NS + s
    e0 = wid * EPW

    bufs = ((colv0, rowv0, rows0, semI0, semG0, semS0),
            (colv1, rowv1, rows1, semI1, semG1, semS1))

    zero16 = jnp.zeros((16,), jnp.float32)
    zero16i = jnp.zeros((16,), jnp.int32)
    iota16 = lax.iota(jnp.int32, 16)

    def idx_load(ci, b):
        colv, rowv, _, semI, _, _ = bufs[b]
        base = e0 + ci * CH
        pltpu.async_copy(row_hbm.at[pl.ds(base, CH)], rowv, semI)
        pltpu.async_copy(col_hbm.at[pl.ds(base, CH)], colv, semI)

    def idx_wait(ci, b):
        colv, rowv, _, semI, _, _ = bufs[b]
        base = e0 + ci * CH
        pltpu.make_async_copy(row_hbm.at[pl.ds(base, CH)], rowv, semI).wait()
        pltpu.make_async_copy(col_hbm.at[pl.ds(base, CH)], colv, semI).wait()

    def gather_start(b):
        colv, _, rows, _, semG, _ = bufs[b]
        pltpu.async_copy(x_hbm.at[colv], rows, semG)

    def gather_wait(b):
        colv, _, rows, _, semG, _ = bufs[b]
        pltpu.make_async_copy(x_hbm.at[colv], rows, semG).wait()

    def scatter_start(b):
        _, rowv, rows, _, _, semS = bufs[b]
        pltpu.async_copy(rows, acc_s.at[rowv], semS, add=True)

    def scatter_wait(b):
        _, rowv, rows, _, _, semS = bufs[b]
        pltpu.make_async_copy(rows, acc_s.at[rowv], semS).wait()

    def hist_update(b):
        _, rowv, _, _, _, _ = bufs[b]
        for g in range(CH // 16):
            rv = rowv[pl.ds(g * 16, 16)]
            hi16 = lax.shift_right_logical(rv, 4)
            lo16 = lax.bitwise_and(rv, 15)
            for j in range(16):
                base16 = hi16[j] * 16
                lo = lo16[j]
                hv = hist[pl.ds(base16, 16)]
                hist[pl.ds(base16, 16)] = hv + jnp.where(iota16 == lo, 1, 0)

    @pl.loop(0, N_NODES // 16)
    def _zero_hist(i):
        hist[pl.ds(i * 16, 16)] = zero16i

    @pl.loop(0, ZR)
    def _fill_zeros(i):
        for j in range(D // 16):
            z[i, pl.ds(j * 16, 16)] = zero16

    row0 = s * RPT

    @pl.loop(0, RPT // ZR)
    def _zero_acc(k):
        off = row0 + k * ZR

        @pl.when(off < N_NODES)
        def _():
            pltpu.sync_copy(z, acc_s.at[pl.ds(off, ZR)])

    plsc.subcore_barrier()

    def step(ci, b):
        idx_wait(ci + 1, 1 - b)
        gather_start(1 - b)
        gather_wait(b)
        scatter_start(b)
        hist_update(b)
        scatter_wait(b)

        @pl.when(ci + 2 < NCHUNK)
        def _():
            idx_load(ci + 2, b)

    idx_load(0, 0)
    idx_load(1, 1)
    idx_wait(0, 0)
    gather_start(0)

    @pl.loop(0, (NCHUNK - 1) // 2)
    def _chunks(k):
        step(2 * k, 0)
        step(2 * k + 1, 1)

    gather_wait(0)
    scatter_start(0)
    hist_update(0)
    scatter_wait(0)

    plsc.subcore_barrier()

    pltpu.sync_copy(hist, cnt_out.at[pl.ds(wid * N_NODES, N_NODES)])

    @pl.when(s < NS - 1)
    def _write_main():
        pltpu.sync_copy(acc_s.at[pl.ds(row0, RPT)],
                        acc_out.at[c, pl.ds(row0, RPT)])

    @pl.when(s == NS - 1)
    def _write_last():
        pltpu.sync_copy(acc_s.at[pl.ds(row0, RPT_LAST)],
                        acc_out.at[c, pl.ds(row0, RPT_LAST)])


BLK = 1000


def _finalize_body(p_ref, c_ref, w_ref, b_ref, o_ref):
    total = p_ref[0] + p_ref[1]
    cnt = jnp.sum(c_ref[...], axis=1, keepdims=True).astype(jnp.float32)
    mean = total / jnp.maximum(cnt, 1.0)
    out = lax.dot_general(
        mean, w_ref[...], (((1,), (1,)), ((), ())),
        preferred_element_type=jnp.float32,
    )
    o_ref[...] = jnp.maximum(out + b_ref[...], 0.0)


_finalize = pl.pallas_call(
    _finalize_body,
    grid=(N_NODES // BLK,),
    in_specs=[
        pl.BlockSpec((NC, BLK, D), lambda i: (0, i, 0)),
        pl.BlockSpec((BLK, NW), lambda i: (i, 0)),
        pl.BlockSpec((D, D), lambda i: (0, 0)),
        pl.BlockSpec((1, D), lambda i: (0, 0)),
    ],
    out_specs=pl.BlockSpec((BLK, D), lambda i: (i, 0)),
    out_shape=jax.ShapeDtypeStruct((N_NODES, D), jnp.float32),
)


def kernel(x, edge_index, W, b):
    row = edge_index[0]
    col = edge_index[1]
    acc, cnt1d = _aggregate(row, col, x)
    cnt_t = cnt1d.reshape(NW, N_NODES).T
    return _finalize(acc, cnt_t, W, b.reshape(1, D))
